# Initial kernel scaffold; baseline (speedup 1.0000x reference)
import jax
import jax.numpy as jnp
from jax.experimental import pallas as pl


def kernel(sequences, ln_gamma, ln_beta, gW, gb, eW, eb, cW, cb):
    raise NotImplementedError("write your pallas kernel here")


# speedup vs baseline: 1.2387x; 1.2387x over previous
import jax
import jax.numpy as jnp
from jax.experimental import pallas as pl
from jax.experimental.pallas import tpu as pltpu

_N, _P, _D = 4, 2048, 1024
_DI = 4096
_E = 8
_TB = 512
_NT = (_N * _P) // _TB


def _body(x_ref, g_ref, b_ref, gWt_ref, gb_ref, eWt_ref, eb_ref, cWt_ref,
          cb_ref, out_ref, h_ref, acc_ref, t1_ref, t2_ref):
    e = pl.program_id(1)

    @pl.when(e == 0)
    def _():
        x = x_ref[...]
        mean = jnp.mean(x, axis=1, keepdims=True)
        xc = x - mean
        var = jnp.mean(xc * xc, axis=1, keepdims=True)
        h = xc * jax.lax.rsqrt(var + 1e-5) * g_ref[...] + b_ref[...]
        h_ref[...] = h.astype(jnp.bfloat16)
        logits = jax.lax.dot_general(
            h, gWt_ref[...], (((1,), (0,)), ((), ())),
            preferred_element_type=jnp.float32) + gb_ref[...]
        neg = jnp.float32(-jnp.inf)
        best1 = jnp.full((_TB, 1), neg, jnp.float32)
        idx1 = jnp.zeros((_TB, 1), jnp.int32)
        for k in range(_E):
            v = logits[:, k:k + 1]
            take = v > best1
            best1 = jnp.where(take, v, best1)
            idx1 = jnp.where(take, k, idx1)
        best2 = jnp.full((_TB, 1), neg, jnp.float32)
        idx2 = jnp.zeros((_TB, 1), jnp.int32)
        for k in range(_E):
            v = jnp.where(idx1 == k, neg, logits[:, k:k + 1])
            take = v > best2
            best2 = jnp.where(take, v, best2)
            idx2 = jnp.where(take, k, idx2)
        t1_ref[...] = idx1
        t2_ref[...] = idx2

    sel = (t1_ref[...] == e) | (t2_ref[...] == e)
    hm = jnp.where(sel, h_ref[...], jnp.bfloat16(0))
    prod = jax.lax.dot_general(hm, eWt_ref[0], (((1,), (0,)), ((), ())),
                               preferred_element_type=jnp.float32)

    @pl.when(e == 0)
    def _():
        acc_ref[...] = prod

    @pl.when(e > 0)
    def _():
        acc_ref[...] += prod

    @pl.when(e == _E - 1)
    def _():
        onehot = jnp.concatenate(
            [((t1_ref[...] == k) | (t2_ref[...] == k)).astype(jnp.float32)
             for k in range(_E)], axis=1)
        bias = jax.lax.dot_general(onehot, eb_ref[...], (((1,), (0,)), ((), ())),
                                   preferred_element_type=jnp.float32)
        act = jnp.maximum(0.5 * (acc_ref[...] + bias), 0.0).astype(jnp.bfloat16)
        cond = jax.lax.dot_general(act, cWt_ref[...], (((1,), (0,)), ((), ())),
                                   preferred_element_type=jnp.float32)
        out_ref[...] = cond + cb_ref[...] + x_ref[...]


def kernel(sequences, ln_gamma, ln_beta, gW, gb, eW, eb, cW, cb):
    x = sequences.reshape(_N * _P, _D)
    gWt = gW.T
    eWt = eW.astype(jnp.bfloat16).swapaxes(1, 2)
    cWt = cW.astype(jnp.bfloat16).T
    out = pl.pallas_call(
        _body,
        grid=(_NT, _E),
        in_specs=[
            pl.BlockSpec((_TB, _D), lambda i, e: (i, 0)),
            pl.BlockSpec((1, _D), lambda i, e: (0, 0)),
            pl.BlockSpec((1, _D), lambda i, e: (0, 0)),
            pl.BlockSpec((_D, _E), lambda i, e: (0, 0)),
            pl.BlockSpec((1, _E), lambda i, e: (0, 0)),
            pl.BlockSpec((1, _D, _DI), lambda i, e: (e, 0, 0)),
            pl.BlockSpec((_E, _DI), lambda i, e: (0, 0)),
            pl.BlockSpec((_DI, _D), lambda i, e: (0, 0)),
            pl.BlockSpec((1, _D), lambda i, e: (0, 0)),
        ],
        out_specs=pl.BlockSpec((_TB, _D), lambda i, e: (i, 0)),
        out_shape=jax.ShapeDtypeStruct((_N * _P, _D), jnp.float32),
        scratch_shapes=[
            pltpu.VMEM((_TB, _D), jnp.bfloat16),
            pltpu.VMEM((_TB, _DI), jnp.float32),
            pltpu.VMEM((_TB, 1), jnp.int32),
            pltpu.VMEM((_TB, 1), jnp.int32),
        ],
    )(x, ln_gamma.reshape(1, _D), ln_beta.reshape(1, _D), gWt,
      gb.reshape(1, _E), eWt, eb, cWt, cb.reshape(1, _D))
    return out.reshape(_N, _P, _D)


# speedup vs baseline: 1.5303x; 1.2354x over previous
import jax
import jax.numpy as jnp
from jax.experimental import pallas as pl
from jax.experimental.pallas import tpu as pltpu
from jax.experimental.pallas import tpu_sc as plsc

_N, _P, _D = 4, 2048, 1024
_DI = 4096
_E = 8
_T = _N * _P
_TB = 512
_NT = _T // _TB
_M = 512
_NB = (2 * _T) // _M + _E
_W = 128
_HC = 4
_HW = _D // _HC
_ZC = 16
_CC = _DI // _ZC


def _prep_body(*refs):
    (x_ref, g_ref, b_ref, gWt_ref, gb_ref) = refs[:5]
    h_refs = refs[5:5 + _HC]
    meta_ref, counts_ref, run_ref, tri_ref = refs[5 + _HC:]
    i = pl.program_id(0)

    @pl.when(i == 0)
    def _():
        run_ref[...] = jnp.zeros_like(run_ref)
        row = jax.lax.broadcasted_iota(jnp.int32, (_TB, _TB), 0)
        col = jax.lax.broadcasted_iota(jnp.int32, (_TB, _TB), 1)
        tri_ref[...] = (col < row).astype(jnp.float32)

    x = x_ref[...]
    mean = jnp.mean(x, axis=1, keepdims=True)
    xc = x - mean
    var = jnp.mean(xc * xc, axis=1, keepdims=True)
    h = xc * jax.lax.rsqrt(var + 1e-5) * g_ref[...] + b_ref[...]
    for c in range(_HC):
        h_refs[c][...] = h[:, c * _HW:(c + 1) * _HW]
    logits = jax.lax.dot_general(
        h, gWt_ref[...], (((1,), (0,)), ((), ())),
        preferred_element_type=jnp.float32) + gb_ref[...]
    neg = jnp.float32(-jnp.inf)
    best1 = jnp.full((_TB, 1), neg, jnp.float32)
    idx1 = jnp.zeros((_TB, 1), jnp.int32)
    for k in range(_E):
        v = logits[:, k:k + 1]
        take = v > best1
        best1 = jnp.where(take, v, best1)
        idx1 = jnp.where(take, k, idx1)
    best2 = jnp.full((_TB, 1), neg, jnp.float32)
    idx2 = jnp.zeros((_TB, 1), jnp.int32)
    for k in range(_E):
        v = jnp.where(idx1 == k, neg, logits[:, k:k + 1])
        take = v > best2
        best2 = jnp.where(take, v, best2)
        idx2 = jnp.where(take, k, idx2)
    mask = jnp.concatenate(
        [((idx1 == k) | (idx2 == k)).astype(jnp.float32) for k in range(_E)],
        axis=1)
    prefix = jax.lax.dot_general(
        tri_ref[...], mask, (((1,), (0,)), ((), ())),
        preferred_element_type=jnp.float32) + run_ref[...].astype(jnp.float32)
    r1 = jnp.zeros((_TB, 1), jnp.float32)
    r2 = jnp.zeros((_TB, 1), jnp.float32)
    for k in range(_E):
        col_k = prefix[:, k:k + 1]
        r1 = jnp.where(idx1 == k, col_k, r1)
        r2 = jnp.where(idx2 == k, col_k, r2)
    zero = jnp.zeros((_TB, 1), jnp.int32)
    meta_ref[...] = jnp.concatenate(
        [idx1, idx2, r1.astype(jnp.int32), r2.astype(jnp.int32),
         zero, zero, zero, zero], axis=1)
    run_ref[...] += jnp.sum(mask, axis=0, keepdims=True).astype(jnp.int32)
    counts_ref[...] = run_ref[...]


def _route_body(counts_ref, meta_ref, pos_ref, be_ref):
    counts = counts_ref[...]
    nb = (counts + (_M - 1)) // _M
    offs = []
    acc = jnp.zeros((), jnp.int32)
    for e in range(_E):
        offs.append(acc)
        acc = acc + nb[0, e]
    meta = meta_ref[...]
    t1 = meta[:, 0:1]
    t2 = meta[:, 1:2]
    r1 = meta[:, 2:3]
    r2 = meta[:, 3:4]
    base1 = jnp.zeros((_TB, 1), jnp.int32)
    base2 = jnp.zeros((_TB, 1), jnp.int32)
    for e in range(_E):
        base_e = _M * offs[e]
        base1 = jnp.where(t1 == e, base_e, base1)
        base2 = jnp.where(t2 == e, base_e, base2)
    p1 = base1 + r1
    p2 = base2 + r2
    pos_ref[0:1, :] = p1.reshape(1, _TB)
    pos_ref[1:2, :] = p2.reshape(1, _TB)
    bio = jax.lax.broadcasted_iota(jnp.int32, (1, _NB), 1)
    be = jnp.zeros((1, _NB), jnp.int32)
    for e in range(1, _E):
        be = be + (bio >= offs[e]).astype(jnp.int32)
    be_ref[...] = be


def _group_mm_body(*refs):
    be_ref = refs[0]
    xs_refs = refs[1:1 + _HC]
    ew_ref, eb_ref = refs[1 + _HC:3 + _HC]
    z_refs = refs[3 + _HC:]
    hcat = jnp.concatenate([r[...] for r in xs_refs],
                           axis=1).astype(jnp.bfloat16)
    prod = jax.lax.dot_general(
        hcat, ew_ref[0], (((1,), (0,)), ((), ())),
        preferred_element_type=jnp.float32)
    z = prod + eb_ref[0]
    for c in range(_ZC):
        z_refs[c][...] = z[:, c * _CC:(c + 1) * _CC]


def _combine_body(*refs):
    za_refs = refs[:_ZC]
    zb_refs = refs[_ZC:2 * _ZC]
    x_ref, cWt_ref, cb_ref, out_ref = refs[2 * _ZC:]
    acts = []
    for c in range(_ZC):
        s = za_refs[c][...] + zb_refs[c][...]
        acts.append(jnp.maximum(0.5 * s, 0.0).astype(jnp.bfloat16))
    act = jnp.concatenate(acts, axis=1)
    cond = jax.lax.dot_general(
        act, cWt_ref[...], (((1,), (0,)), ((), ())),
        preferred_element_type=jnp.float32)
    out_ref[...] = cond + cb_ref[...] + x_ref[...]


def _vector_mesh():
    return plsc.VectorSubcoreMesh(core_axis_name="c", subcore_axis_name="s")


def _sc_scatter(h_chunks, pos):
    nh = _T // _W
    out_ty = [jax.ShapeDtypeStruct((_NB * _M, _HW), jnp.float32)
              for _ in range(_HC)]

    @pl.kernel(out_type=out_ty, mesh=_vector_mesh())
    def run(*refs):
        h_hbms = refs[:_HC]
        pos_hbm = refs[_HC]
        o_hbms = refs[_HC + 1:]
        for c in range(_HC):
            def body(h_vmem, i_vmem, o_hbm=o_hbms[c]):
                pltpu.sync_copy(h_vmem, o_hbm.at[i_vmem.at[0]])

            pltpu.emit_pipeline(
                body,
                grid=(2, nh),
                in_specs=[pl.BlockSpec((_W, _HW), lambda i, j: (j, 0)),
                          pl.BlockSpec((1, _W), lambda i, j: (i, j))],
                out_specs=[],
                core_axis_name=("c", "s"),
                dimension_semantics=(pltpu.PARALLEL, pltpu.PARALLEL),
            )(h_hbms[c], pos_hbm)

    return run(*h_chunks, pos)


def _sc_gather(z_chunks, pos):
    ng = _T // _W
    out_ty = [jax.ShapeDtypeStruct((2 * _T, _CC), jnp.float32)
              for _ in range(_ZC)]

    @pl.kernel(out_type=out_ty, mesh=_vector_mesh())
    def run(*refs):
        z_hbms = refs[:_ZC]
        pos_hbm = refs[_ZC]
        o_hbms = refs[_ZC + 1:]
        for c in range(_ZC):
            def body(i_vmem, o_vmem, z_hbm=z_hbms[c]):
                pltpu.sync_copy(z_hbm.at[i_vmem.at[0]], o_vmem)

            pltpu.emit_pipeline(
                body,
                grid=(2, ng),
                in_specs=[pl.BlockSpec((1, _W), lambda i, j: (i, j))],
                out_specs=[pl.BlockSpec((_W, _CC),
                                        lambda i, j: (i * ng + j, 0))],
                core_axis_name=("c", "s"),
                dimension_semantics=(pltpu.PARALLEL, pltpu.PARALLEL),
            )(pos_hbm, o_hbms[c])

    return run(*z_chunks, pos)


def kernel(sequences, ln_gamma, ln_beta, gW, gb, eW, eb, cW, cb):
    x = sequences.reshape(_T, _D)
    gWt = gW.T
    eWt = eW.astype(jnp.bfloat16).swapaxes(1, 2)
    cWt = cW.astype(jnp.bfloat16).T

    outs = pl.pallas_call(
        _prep_body,
        grid=(_NT,),
        in_specs=[
            pl.BlockSpec((_TB, _D), lambda i: (i, 0)),
            pl.BlockSpec((1, _D), lambda i: (0, 0)),
            pl.BlockSpec((1, _D), lambda i: (0, 0)),
            pl.BlockSpec((_D, _E), lambda i: (0, 0)),
            pl.BlockSpec((1, _E), lambda i: (0, 0)),
        ],
        out_specs=(
            [pl.BlockSpec((_TB, _HW), lambda i: (i, 0))
             for _ in range(_HC)] +
            [pl.BlockSpec((_TB, _E), lambda i: (i, 0)),
             pl.BlockSpec((1, _E), lambda i: (0, 0))]),
        out_shape=(
            [jax.ShapeDtypeStruct((_T, _HW), jnp.float32)
             for _ in range(_HC)] +
            [jax.ShapeDtypeStruct((_T, _E), jnp.int32),
             jax.ShapeDtypeStruct((1, _E), jnp.int32)]),
        scratch_shapes=[
            pltpu.VMEM((1, _E), jnp.int32),
            pltpu.VMEM((_TB, _TB), jnp.float32),
        ],
    )(x, ln_gamma.reshape(1, _D), ln_beta.reshape(1, _D), gWt,
      gb.reshape(1, _E))
    h_chunks = outs[:_HC]
    meta, counts = outs[_HC:]

    pos, be = pl.pallas_call(
        _route_body,
        grid=(_NT,),
        in_specs=[
            pl.BlockSpec((1, _E), lambda i: (0, 0)),
            pl.BlockSpec((_TB, _E), lambda i: (i, 0)),
        ],
        out_specs=[
            pl.BlockSpec((2, _TB), lambda i: (0, i)),
            pl.BlockSpec((1, _NB), lambda i: (0, 0)),
        ],
        out_shape=[
            jax.ShapeDtypeStruct((2, _T), jnp.int32),
            jax.ShapeDtypeStruct((1, _NB), jnp.int32),
        ],
    )(counts, meta)

    xs_chunks = _sc_scatter(h_chunks, pos)

    z_chunks = pl.pallas_call(
        _group_mm_body,
        grid_spec=pltpu.PrefetchScalarGridSpec(
            num_scalar_prefetch=1,
            grid=(_NB,),
            in_specs=(
                [pl.BlockSpec((_M, _HW), lambda b, be_s: (b, 0))
                 for _ in range(_HC)] +
                [pl.BlockSpec((1, _D, _DI), lambda b, be_s: (be_s[b], 0, 0)),
                 pl.BlockSpec((1, 1, _DI), lambda b, be_s: (be_s[b], 0, 0))]),
            out_specs=[pl.BlockSpec((_M, _CC), lambda b, be_s: (b, 0))
                       for _ in range(_ZC)],
        ),
        out_shape=[jax.ShapeDtypeStruct((_NB * _M, _CC), jnp.float32)
                   for _ in range(_ZC)],
    )(be.reshape(_NB), *xs_chunks, eWt, eb.reshape(_E, 1, _DI))

    zg_chunks = _sc_gather(z_chunks, pos)

    out = pl.pallas_call(
        _combine_body,
        grid=(_NT,),
        in_specs=(
            [pl.BlockSpec((_TB, _CC), lambda i: (i, 0))
             for _ in range(_ZC)] +
            [pl.BlockSpec((_TB, _CC), lambda i: (i + _NT, 0))
             for _ in range(_ZC)] +
            [
                pl.BlockSpec((_TB, _D), lambda i: (i, 0)),
                pl.BlockSpec((_DI, _D), lambda i: (0, 0)),
                pl.BlockSpec((1, _D), lambda i: (0, 0)),
            ]),
        out_specs=pl.BlockSpec((_TB, _D), lambda i: (i, 0)),
        out_shape=jax.ShapeDtypeStruct((_T, _D), jnp.float32),
    )(*zg_chunks, *zg_chunks, x, cWt, cb.reshape(1, _D))

    return out.reshape(_N, _P, _D)


# speedup vs baseline: 1.8967x; 1.2395x over previous
import jax
import jax.numpy as jnp
from jax.experimental import pallas as pl
from jax.experimental.pallas import tpu as pltpu
from jax.experimental.pallas import tpu_sc as plsc

_N, _P, _D = 4, 2048, 1024
_DI = 4096
_E = 8
_T = _N * _P
_TB = 512
_NT = _T // _TB
_M = 512
_NB = (2 * _T) // _M + _E
_W = 128
_HC = 2
_HW = _D // 2 // _HC
_ZC = 8
_CC = _DI // 2 // _ZC


def _pack_pairs(vb):
    n = vb.shape[1] // 2
    lo = pltpu.bitcast(vb[:, :n], jnp.uint16).astype(jnp.uint32)
    hi = pltpu.bitcast(vb[:, n:], jnp.uint16).astype(jnp.uint32)
    return pltpu.bitcast(lo | (hi << 16), jnp.int32)


def _unpack_pairs(w):
    u = pltpu.bitcast(w, jnp.uint32)
    lo = pltpu.bitcast((u & 0xFFFF).astype(jnp.uint16), jnp.bfloat16)
    hi = pltpu.bitcast((u >> 16).astype(jnp.uint16), jnp.bfloat16)
    return lo, hi


def _prep_body(*refs):
    (x_ref, g_ref, b_ref, gWt_ref, gb_ref) = refs[:5]
    h_refs = refs[5:5 + _HC]
    meta_ref, counts_ref, run_ref, tri_ref = refs[5 + _HC:]
    i = pl.program_id(0)

    @pl.when(i == 0)
    def _():
        run_ref[...] = jnp.zeros_like(run_ref)
        row = jax.lax.broadcasted_iota(jnp.int32, (_TB, _TB), 0)
        col = jax.lax.broadcasted_iota(jnp.int32, (_TB, _TB), 1)
        tri_ref[...] = (col < row).astype(jnp.float32)

    x = x_ref[...]
    mean = jnp.mean(x, axis=1, keepdims=True)
    xc = x - mean
    var = jnp.mean(xc * xc, axis=1, keepdims=True)
    h = xc * jax.lax.rsqrt(var + 1e-5) * g_ref[...] + b_ref[...]
    hw = _pack_pairs(h.astype(jnp.bfloat16))
    for c in range(_HC):
        h_refs[c][...] = hw[:, c * _HW:(c + 1) * _HW]
    logits = jax.lax.dot_general(
        h, gWt_ref[...], (((1,), (0,)), ((), ())),
        preferred_element_type=jnp.float32) + gb_ref[...]
    neg = jnp.float32(-jnp.inf)
    best1 = jnp.full((_TB, 1), neg, jnp.float32)
    idx1 = jnp.zeros((_TB, 1), jnp.int32)
    for k in range(_E):
        v = logits[:, k:k + 1]
        take = v > best1
        best1 = jnp.where(take, v, best1)
        idx1 = jnp.where(take, k, idx1)
    best2 = jnp.full((_TB, 1), neg, jnp.float32)
    idx2 = jnp.zeros((_TB, 1), jnp.int32)
    for k in range(_E):
        v = jnp.where(idx1 == k, neg, logits[:, k:k + 1])
        take = v > best2
        best2 = jnp.where(take, v, best2)
        idx2 = jnp.where(take, k, idx2)
    mask = jnp.concatenate(
        [((idx1 == k) | (idx2 == k)).astype(jnp.float32) for k in range(_E)],
        axis=1)
    prefix = jax.lax.dot_general(
        tri_ref[...], mask, (((1,), (0,)), ((), ())),
        preferred_element_type=jnp.float32) + run_ref[...].astype(jnp.float32)
    r1 = jnp.zeros((_TB, 1), jnp.float32)
    r2 = jnp.zeros((_TB, 1), jnp.float32)
    for k in range(_E):
        col_k = prefix[:, k:k + 1]
        r1 = jnp.where(idx1 == k, col_k, r1)
        r2 = jnp.where(idx2 == k, col_k, r2)
    zero = jnp.zeros((_TB, 1), jnp.int32)
    meta_ref[...] = jnp.concatenate(
        [idx1, idx2, r1.astype(jnp.int32), r2.astype(jnp.int32),
         zero, zero, zero, zero], axis=1)
    run_ref[...] += jnp.sum(mask, axis=0, keepdims=True).astype(jnp.int32)
    counts_ref[...] = run_ref[...]


def _route_body(counts_ref, meta_ref, pos_ref, be_ref):
    counts = counts_ref[...]
    nb = (counts + (_M - 1)) // _M
    offs = []
    acc = jnp.zeros((), jnp.int32)
    for e in range(_E):
        offs.append(acc)
        acc = acc + nb[0, e]
    meta = meta_ref[...]
    t1 = meta[:, 0:1]
    t2 = meta[:, 1:2]
    r1 = meta[:, 2:3]
    r2 = meta[:, 3:4]
    base1 = jnp.zeros((_TB, 1), jnp.int32)
    base2 = jnp.zeros((_TB, 1), jnp.int32)
    for e in range(_E):
        base_e = _M * offs[e]
        base1 = jnp.where(t1 == e, base_e, base1)
        base2 = jnp.where(t2 == e, base_e, base2)
    p1 = base1 + r1
    p2 = base2 + r2
    pos_ref[0:1, :] = p1.reshape(1, _TB)
    pos_ref[1:2, :] = p2.reshape(1, _TB)
    bio = jax.lax.broadcasted_iota(jnp.int32, (1, _NB), 1)
    be = jnp.zeros((1, _NB), jnp.int32)
    for e in range(1, _E):
        be = be + (bio >= offs[e]).astype(jnp.int32)
    be_ref[...] = be


def _group_mm_body(*refs):
    be_ref = refs[0]
    xs_refs = refs[1:1 + _HC]
    ew_ref, eb_ref = refs[1 + _HC:3 + _HC]
    z_refs = refs[3 + _HC:]
    lows, highs = [], []
    for r in xs_refs:
        lo, hi = _unpack_pairs(r[...])
        lows.append(lo)
        highs.append(hi)
    hcat = jnp.concatenate(lows + highs, axis=1)
    prod = jax.lax.dot_general(
        hcat, ew_ref[0], (((1,), (0,)), ((), ())),
        preferred_element_type=jnp.float32)
    zw = _pack_pairs((prod + eb_ref[0]).astype(jnp.bfloat16))
    for c in range(_ZC):
        z_refs[c][...] = zw[:, c * _CC:(c + 1) * _CC]


def _combine_body(*refs):
    za_refs = refs[:_ZC]
    zb_refs = refs[_ZC:2 * _ZC]
    x_ref, cWt_ref, cb_ref, out_ref = refs[2 * _ZC:]
    acts_lo, acts_hi = [], []
    for c in range(_ZC):
        a_lo, a_hi = _unpack_pairs(za_refs[c][...])
        b_lo, b_hi = _unpack_pairs(zb_refs[c][...])
        s_lo = a_lo.astype(jnp.float32) + b_lo.astype(jnp.float32)
        s_hi = a_hi.astype(jnp.float32) + b_hi.astype(jnp.float32)
        acts_lo.append(jnp.maximum(0.5 * s_lo, 0.0).astype(jnp.bfloat16))
        acts_hi.append(jnp.maximum(0.5 * s_hi, 0.0).astype(jnp.bfloat16))
    act = jnp.concatenate(acts_lo + acts_hi, axis=1)
    cond = jax.lax.dot_general(
        act, cWt_ref[...], (((1,), (0,)), ((), ())),
        preferred_element_type=jnp.float32)
    out_ref[...] = cond + cb_ref[...] + x_ref[...]


def _vector_mesh():
    return plsc.VectorSubcoreMesh(core_axis_name="c", subcore_axis_name="s")


def _sc_scatter(h_chunks, pos):
    nh = _T // _W
    out_ty = [jax.ShapeDtypeStruct((_NB * _M, _HW), jnp.int32)
              for _ in range(_HC)]

    @pl.kernel(out_type=out_ty, mesh=_vector_mesh())
    def run(*refs):
        h_hbms = refs[:_HC]
        pos_hbm = refs[_HC]
        o_hbms = refs[_HC + 1:]
        for c in range(_HC):
            def body(h_vmem, i_vmem, o_hbm=o_hbms[c]):
                pltpu.sync_copy(h_vmem, o_hbm.at[i_vmem.at[0]])

            pltpu.emit_pipeline(
                body,
                grid=(2, nh),
                in_specs=[pl.BlockSpec((_W, _HW), lambda i, j: (j, 0)),
                          pl.BlockSpec((1, _W), lambda i, j: (i, j))],
                out_specs=[],
                core_axis_name=("c", "s"),
                dimension_semantics=(pltpu.PARALLEL, pltpu.PARALLEL),
            )(h_hbms[c], pos_hbm)

    return run(*h_chunks, pos)


def _sc_gather(z_chunks, pos):
    ng = _T // _W
    out_ty = [jax.ShapeDtypeStruct((2 * _T, _CC), jnp.int32)
              for _ in range(_ZC)]

    @pl.kernel(out_type=out_ty, mesh=_vector_mesh())
    def run(*refs):
        z_hbms = refs[:_ZC]
        pos_hbm = refs[_ZC]
        o_hbms = refs[_ZC + 1:]
        for c in range(_ZC):
            def body(i_vmem, o_vmem, z_hbm=z_hbms[c]):
                pltpu.sync_copy(z_hbm.at[i_vmem.at[0]], o_vmem)

            pltpu.emit_pipeline(
                body,
                grid=(2, ng),
                in_specs=[pl.BlockSpec((1, _W), lambda i, j: (i, j))],
                out_specs=[pl.BlockSpec((_W, _CC),
                                        lambda i, j: (i * ng + j, 0))],
                core_axis_name=("c", "s"),
                dimension_semantics=(pltpu.PARALLEL, pltpu.PARALLEL),
            )(pos_hbm, o_hbms[c])

    return run(*z_chunks, pos)


def kernel(sequences, ln_gamma, ln_beta, gW, gb, eW, eb, cW, cb):
    x = sequences.reshape(_T, _D)
    gWt = gW.T
    eWt = eW.astype(jnp.bfloat16).swapaxes(1, 2)
    cWt = cW.astype(jnp.bfloat16).T

    outs = pl.pallas_call(
        _prep_body,
        grid=(_NT,),
        in_specs=[
            pl.BlockSpec((_TB, _D), lambda i: (i, 0)),
            pl.BlockSpec((1, _D), lambda i: (0, 0)),
            pl.BlockSpec((1, _D), lambda i: (0, 0)),
            pl.BlockSpec((_D, _E), lambda i: (0, 0)),
            pl.BlockSpec((1, _E), lambda i: (0, 0)),
        ],
        out_specs=(
            [pl.BlockSpec((_TB, _HW), lambda i: (i, 0))
             for _ in range(_HC)] +
            [pl.BlockSpec((_TB, _E), lambda i: (i, 0)),
             pl.BlockSpec((1, _E), lambda i: (0, 0))]),
        out_shape=(
            [jax.ShapeDtypeStruct((_T, _HW), jnp.int32)
             for _ in range(_HC)] +
            [jax.ShapeDtypeStruct((_T, _E), jnp.int32),
             jax.ShapeDtypeStruct((1, _E), jnp.int32)]),
        scratch_shapes=[
            pltpu.VMEM((1, _E), jnp.int32),
            pltpu.VMEM((_TB, _TB), jnp.float32),
        ],
    )(x, ln_gamma.reshape(1, _D), ln_beta.reshape(1, _D), gWt,
      gb.reshape(1, _E))
    h_chunks = outs[:_HC]
    meta, counts = outs[_HC:]

    pos, be = pl.pallas_call(
        _route_body,
        grid=(_NT,),
        in_specs=[
            pl.BlockSpec((1, _E), lambda i: (0, 0)),
            pl.BlockSpec((_TB, _E), lambda i: (i, 0)),
        ],
        out_specs=[
            pl.BlockSpec((2, _TB), lambda i: (0, i)),
            pl.BlockSpec((1, _NB), lambda i: (0, 0)),
        ],
        out_shape=[
            jax.ShapeDtypeStruct((2, _T), jnp.int32),
            jax.ShapeDtypeStruct((1, _NB), jnp.int32),
        ],
    )(counts, meta)

    xs_chunks = _sc_scatter(h_chunks, pos)

    z_chunks = pl.pallas_call(
        _group_mm_body,
        grid_spec=pltpu.PrefetchScalarGridSpec(
            num_scalar_prefetch=1,
            grid=(_NB,),
            in_specs=(
                [pl.BlockSpec((_M, _HW), lambda b, be_s: (b, 0))
                 for _ in range(_HC)] +
                [pl.BlockSpec((1, _D, _DI), lambda b, be_s: (be_s[b], 0, 0)),
                 pl.BlockSpec((1, 1, _DI), lambda b, be_s: (be_s[b], 0, 0))]),
            out_specs=[pl.BlockSpec((_M, _CC), lambda b, be_s: (b, 0))
                       for _ in range(_ZC)],
        ),
        out_shape=[jax.ShapeDtypeStruct((_NB * _M, _CC), jnp.int32)
                   for _ in range(_ZC)],
    )(be.reshape(_NB), *xs_chunks, eWt, eb.reshape(_E, 1, _DI))

    zg_chunks = _sc_gather(z_chunks, pos)

    out = pl.pallas_call(
        _combine_body,
        grid=(_NT,),
        in_specs=(
            [pl.BlockSpec((_TB, _CC), lambda i: (i, 0))
             for _ in range(_ZC)] +
            [pl.BlockSpec((_TB, _CC), lambda i: (i + _NT, 0))
             for _ in range(_ZC)] +
            [
                pl.BlockSpec((_TB, _D), lambda i: (i, 0)),
                pl.BlockSpec((_DI, _D), lambda i: (0, 0)),
                pl.BlockSpec((1, _D), lambda i: (0, 0)),
            ]),
        out_specs=pl.BlockSpec((_TB, _D), lambda i: (i, 0)),
        out_shape=jax.ShapeDtypeStruct((_T, _D), jnp.float32),
    )(*zg_chunks, *zg_chunks, x, cWt, cb.reshape(1, _D))

    return out.reshape(_N, _P, _D)


# speedup vs baseline: 1.9703x; 1.0388x over previous
import jax
import jax.numpy as jnp
from jax.experimental import pallas as pl
from jax.experimental.pallas import tpu as pltpu
from jax.experimental.pallas import tpu_sc as plsc

_N, _P, _D = 4, 2048, 1024
_DI = 4096
_E = 8
_T = _N * _P
_TB = 512
_NT = _T // _TB
_M = 512
_NB = (2 * _T) // _M + _E
_W = 128
_HC = 2
_HW = _D // 2 // _HC
_ZC = 8
_CC = _DI // 2 // _ZC


def _pack_pairs(vb):
    n = vb.shape[1] // 2
    lo = pltpu.bitcast(vb[:, :n], jnp.uint16).astype(jnp.uint32)
    hi = pltpu.bitcast(vb[:, n:], jnp.uint16).astype(jnp.uint32)
    return pltpu.bitcast(lo | (hi << 16), jnp.int32)


def _unpack_pairs(w):
    u = pltpu.bitcast(w, jnp.uint32)
    lo = pltpu.bitcast((u & 0xFFFF).astype(jnp.uint16), jnp.bfloat16)
    hi = pltpu.bitcast((u >> 16).astype(jnp.uint16), jnp.bfloat16)
    return lo, hi


def _prep_body(*refs):
    (x_ref, g_ref, b_ref, gWt_ref, gb_ref) = refs[:5]
    h_refs = refs[5:5 + _HC]
    meta_ref, counts_ref, run_ref, tri_ref = refs[5 + _HC:]
    i = pl.program_id(0)

    @pl.when(i == 0)
    def _():
        run_ref[...] = jnp.zeros_like(run_ref)
        row = jax.lax.broadcasted_iota(jnp.int32, (_TB, _TB), 0)
        col = jax.lax.broadcasted_iota(jnp.int32, (_TB, _TB), 1)
        tri_ref[...] = (col < row).astype(jnp.float32)

    x = x_ref[...]
    mean = jnp.mean(x, axis=1, keepdims=True)
    xc = x - mean
    var = jnp.mean(xc * xc, axis=1, keepdims=True)
    h = xc * jax.lax.rsqrt(var + 1e-5) * g_ref[...] + b_ref[...]
    hw = _pack_pairs(h.astype(jnp.bfloat16))
    for c in range(_HC):
        h_refs[c][...] = hw[:, c * _HW:(c + 1) * _HW]
    logits = jax.lax.dot_general(
        h, gWt_ref[...], (((1,), (0,)), ((), ())),
        preferred_element_type=jnp.float32) + gb_ref[...]
    neg = jnp.float32(-jnp.inf)
    best1 = jnp.full((_TB, 1), neg, jnp.float32)
    idx1 = jnp.zeros((_TB, 1), jnp.int32)
    for k in range(_E):
        v = logits[:, k:k + 1]
        take = v > best1
        best1 = jnp.where(take, v, best1)
        idx1 = jnp.where(take, k, idx1)
    best2 = jnp.full((_TB, 1), neg, jnp.float32)
    idx2 = jnp.zeros((_TB, 1), jnp.int32)
    for k in range(_E):
        v = jnp.where(idx1 == k, neg, logits[:, k:k + 1])
        take = v > best2
        best2 = jnp.where(take, v, best2)
        idx2 = jnp.where(take, k, idx2)
    mask = jnp.concatenate(
        [((idx1 == k) | (idx2 == k)).astype(jnp.float32) for k in range(_E)],
        axis=1)
    prefix = jax.lax.dot_general(
        tri_ref[...], mask, (((1,), (0,)), ((), ())),
        preferred_element_type=jnp.float32) + run_ref[...].astype(jnp.float32)
    r1 = jnp.zeros((_TB, 1), jnp.float32)
    r2 = jnp.zeros((_TB, 1), jnp.float32)
    for k in range(_E):
        col_k = prefix[:, k:k + 1]
        r1 = jnp.where(idx1 == k, col_k, r1)
        r2 = jnp.where(idx2 == k, col_k, r2)
    zero = jnp.zeros((_TB, 1), jnp.int32)
    meta_ref[...] = jnp.concatenate(
        [idx1, idx2, r1.astype(jnp.int32), r2.astype(jnp.int32),
         zero, zero, zero, zero], axis=1)
    run_ref[...] += jnp.sum(mask, axis=0, keepdims=True).astype(jnp.int32)
    counts_ref[...] = run_ref[...]


def _route_body(counts_ref, meta_ref, pos_ref, be_ref):
    counts = counts_ref[...]
    nb = (counts + (_M - 1)) // _M
    offs = []
    acc = jnp.zeros((), jnp.int32)
    for e in range(_E):
        offs.append(acc)
        acc = acc + nb[0, e]
    meta = meta_ref[...]
    t1 = meta[:, 0:1]
    t2 = meta[:, 1:2]
    r1 = meta[:, 2:3]
    r2 = meta[:, 3:4]
    base1 = jnp.zeros((_TB, 1), jnp.int32)
    base2 = jnp.zeros((_TB, 1), jnp.int32)
    for e in range(_E):
        base_e = _M * offs[e]
        base1 = jnp.where(t1 == e, base_e, base1)
        base2 = jnp.where(t2 == e, base_e, base2)
    p1 = base1 + r1
    p2 = base2 + r2
    pos_ref[0:1, :] = p1.reshape(1, _TB)
    pos_ref[1:2, :] = p2.reshape(1, _TB)
    bio = jax.lax.broadcasted_iota(jnp.int32, (1, _NB), 1)
    be = jnp.zeros((1, _NB), jnp.int32)
    for e in range(1, _E):
        be = be + (bio >= offs[e]).astype(jnp.int32)
    be_ref[...] = be


def _group_mm_body(*refs):
    be_ref = refs[0]
    xs_refs = refs[1:1 + _HC]
    ew_ref, eb_ref = refs[1 + _HC:3 + _HC]
    z_refs = refs[3 + _HC:]
    lows, highs = [], []
    for r in xs_refs:
        lo, hi = _unpack_pairs(r[...])
        lows.append(lo)
        highs.append(hi)
    hcat = jnp.concatenate(lows + highs, axis=1)
    prod = jax.lax.dot_general(
        hcat, ew_ref[0], (((1,), (1,)), ((), ())),
        preferred_element_type=jnp.float32)
    zw = _pack_pairs((prod + eb_ref[0]).astype(jnp.bfloat16))
    for c in range(_ZC):
        z_refs[c][...] = zw[:, c * _CC:(c + 1) * _CC]


def _combine_body(*refs):
    za_refs = refs[:_ZC]
    zb_refs = refs[_ZC:2 * _ZC]
    x_ref, cWt_ref, cb_ref, out_ref = refs[2 * _ZC:]
    acts_lo, acts_hi = [], []
    for c in range(_ZC):
        a_lo, a_hi = _unpack_pairs(za_refs[c][...])
        b_lo, b_hi = _unpack_pairs(zb_refs[c][...])
        s_lo = a_lo.astype(jnp.float32) + b_lo.astype(jnp.float32)
        s_hi = a_hi.astype(jnp.float32) + b_hi.astype(jnp.float32)
        acts_lo.append(jnp.maximum(0.5 * s_lo, 0.0).astype(jnp.bfloat16))
        acts_hi.append(jnp.maximum(0.5 * s_hi, 0.0).astype(jnp.bfloat16))
    act = jnp.concatenate(acts_lo + acts_hi, axis=1)
    cond = jax.lax.dot_general(
        act, cWt_ref[...], (((1,), (1,)), ((), ())),
        preferred_element_type=jnp.float32)
    out_ref[...] = cond + cb_ref[...] + x_ref[...]


def _vector_mesh():
    return plsc.VectorSubcoreMesh(core_axis_name="c", subcore_axis_name="s")


def _sc_scatter(h_chunks, pos):
    nh = _T // _W
    out_ty = [jax.ShapeDtypeStruct((_NB * _M, _HW), jnp.int32)
              for _ in range(_HC)]

    @pl.kernel(out_type=out_ty, mesh=_vector_mesh())
    def run(*refs):
        h_hbms = refs[:_HC]
        pos_hbm = refs[_HC]
        o_hbms = refs[_HC + 1:]
        for c in range(_HC):
            def body(h_vmem, i_vmem, o_hbm=o_hbms[c]):
                pltpu.sync_copy(h_vmem, o_hbm.at[i_vmem.at[0]])

            pltpu.emit_pipeline(
                body,
                grid=(2, nh),
                in_specs=[pl.BlockSpec((_W, _HW), lambda i, j: (j, 0)),
                          pl.BlockSpec((1, _W), lambda i, j: (i, j))],
                out_specs=[],
                core_axis_name=("c", "s"),
                dimension_semantics=(pltpu.PARALLEL, pltpu.PARALLEL),
            )(h_hbms[c], pos_hbm)

    return run(*h_chunks, pos)


def _sc_gather(z_chunks, pos):
    ng = _T // _W
    out_ty = [jax.ShapeDtypeStruct((2 * _T, _CC), jnp.int32)
              for _ in range(_ZC)]

    @pl.kernel(out_type=out_ty, mesh=_vector_mesh())
    def run(*refs):
        z_hbms = refs[:_ZC]
        pos_hbm = refs[_ZC]
        o_hbms = refs[_ZC + 1:]
        for c in range(_ZC):
            def body(i_vmem, o_vmem, z_hbm=z_hbms[c]):
                pltpu.sync_copy(z_hbm.at[i_vmem.at[0]], o_vmem)

            pltpu.emit_pipeline(
                body,
                grid=(2, ng),
                in_specs=[pl.BlockSpec((1, _W), lambda i, j: (i, j))],
                out_specs=[pl.BlockSpec((_W, _CC),
                                        lambda i, j: (i * ng + j, 0))],
                core_axis_name=("c", "s"),
                dimension_semantics=(pltpu.PARALLEL, pltpu.PARALLEL),
            )(pos_hbm, o_hbms[c])

    return run(*z_chunks, pos)


def kernel(sequences, ln_gamma, ln_beta, gW, gb, eW, eb, cW, cb):
    x = sequences.reshape(_T, _D)
    gWt = gW.T
    eWt = eW.astype(jnp.bfloat16)
    cWt = cW.astype(jnp.bfloat16)

    outs = pl.pallas_call(
        _prep_body,
        grid=(_NT,),
        in_specs=[
            pl.BlockSpec((_TB, _D), lambda i: (i, 0)),
            pl.BlockSpec((1, _D), lambda i: (0, 0)),
            pl.BlockSpec((1, _D), lambda i: (0, 0)),
            pl.BlockSpec((_D, _E), lambda i: (0, 0)),
            pl.BlockSpec((1, _E), lambda i: (0, 0)),
        ],
        out_specs=(
            [pl.BlockSpec((_TB, _HW), lambda i: (i, 0))
             for _ in range(_HC)] +
            [pl.BlockSpec((_TB, _E), lambda i: (i, 0)),
             pl.BlockSpec((1, _E), lambda i: (0, 0))]),
        out_shape=(
            [jax.ShapeDtypeStruct((_T, _HW), jnp.int32)
             for _ in range(_HC)] +
            [jax.ShapeDtypeStruct((_T, _E), jnp.int32),
             jax.ShapeDtypeStruct((1, _E), jnp.int32)]),
        scratch_shapes=[
            pltpu.VMEM((1, _E), jnp.int32),
            pltpu.VMEM((_TB, _TB), jnp.float32),
        ],
    )(x, ln_gamma.reshape(1, _D), ln_beta.reshape(1, _D), gWt,
      gb.reshape(1, _E))
    h_chunks = outs[:_HC]
    meta, counts = outs[_HC:]

    pos, be = pl.pallas_call(
        _route_body,
        grid=(_NT,),
        in_specs=[
            pl.BlockSpec((1, _E), lambda i: (0, 0)),
            pl.BlockSpec((_TB, _E), lambda i: (i, 0)),
        ],
        out_specs=[
            pl.BlockSpec((2, _TB), lambda i: (0, i)),
            pl.BlockSpec((1, _NB), lambda i: (0, 0)),
        ],
        out_shape=[
            jax.ShapeDtypeStruct((2, _T), jnp.int32),
            jax.ShapeDtypeStruct((1, _NB), jnp.int32),
        ],
    )(counts, meta)

    xs_chunks = _sc_scatter(h_chunks, pos)

    z_chunks = pl.pallas_call(
        _group_mm_body,
        grid_spec=pltpu.PrefetchScalarGridSpec(
            num_scalar_prefetch=1,
            grid=(_NB,),
            in_specs=(
                [pl.BlockSpec((_M, _HW), lambda b, be_s: (b, 0))
                 for _ in range(_HC)] +
                [pl.BlockSpec((1, _DI, _D), lambda b, be_s: (be_s[b], 0, 0)),
                 pl.BlockSpec((1, 1, _DI), lambda b, be_s: (be_s[b], 0, 0))]),
            out_specs=[pl.BlockSpec((_M, _CC), lambda b, be_s: (b, 0))
                       for _ in range(_ZC)],
        ),
        out_shape=[jax.ShapeDtypeStruct((_NB * _M, _CC), jnp.int32)
                   for _ in range(_ZC)],
    )(be.reshape(_NB), *xs_chunks, eWt, eb.reshape(_E, 1, _DI))

    zg_chunks = _sc_gather(z_chunks, pos)

    out = pl.pallas_call(
        _combine_body,
        grid=(_NT,),
        in_specs=(
            [pl.BlockSpec((_TB, _CC), lambda i: (i, 0))
             for _ in range(_ZC)] +
            [pl.BlockSpec((_TB, _CC), lambda i: (i + _NT, 0))
             for _ in range(_ZC)] +
            [
                pl.BlockSpec((_TB, _D), lambda i: (i, 0)),
                pl.BlockSpec((_D, _DI), lambda i: (0, 0)),
                pl.BlockSpec((1, _D), lambda i: (0, 0)),
            ]),
        out_specs=pl.BlockSpec((_TB, _D), lambda i: (i, 0)),
        out_shape=jax.ShapeDtypeStruct((_T, _D), jnp.float32),
    )(*zg_chunks, *zg_chunks, x, cWt, cb.reshape(1, _D))

    return out.reshape(_N, _P, _D)
